# Initial kernel scaffold; baseline (speedup 1.0000x reference)
import jax
import jax.numpy as jnp
from jax.experimental import pallas as pl


def kernel(feat0, feat1, feat2, feat3, W_enc, b_enc, ln_g, ln_b, W_cls, b_cls, Wb1, bb1, Wb2, bb2, Wb3, bb3):
    raise NotImplementedError("write your pallas kernel here")


# speedup vs baseline: 1.2075x; 1.2075x over previous
import functools

import numpy as np
import jax
import jax.numpy as jnp
from jax import lax
from jax.experimental import pallas as pl
from jax.experimental.pallas import tpu as pltpu
from jax.experimental.pallas import tpu_sc as plsc

_HID = 256
_NCLS = 91
_NQ = 900
_SHAPES = ((16, 16), (32, 32), (64, 64), (128, 128))
_LVL_N = (256, 1024, 4096, 16384)
_L_TOT = 21760
_TBLK = 256
_NBLK = (1, 4, 16, 64)
_BLK_OFF = (0, 1, 5, 21)
_GRID_I = 85
_NEG = -3.0e38


def _proposals_np():
    props = []
    for lvl, (H, W) in enumerate(_SHAPES):
        gy, gx = np.meshgrid(
            np.linspace(0.0, H - 1.0, H, dtype=np.float32),
            np.linspace(0.0, W - 1.0, W, dtype=np.float32),
            indexing="ij",
        )
        grid = np.stack([gx, gy], axis=-1)
        grid = (grid + 0.5) / np.array([W, H], dtype=np.float32)
        wh = np.ones_like(grid) * np.float32(0.05 * (2.0 ** lvl))
        props.append(np.concatenate([grid, wh], axis=-1).reshape(H * W, 4))
    op = np.concatenate(props, axis=0).astype(np.float32)
    valid = np.all((op > 0.01) & (op < 0.99), axis=-1)
    with np.errstate(divide="ignore"):
        logit = np.log(op / (1.0 - op)).astype(np.float32)
    op_out = np.where(valid[:, None], logit, np.float32(np.inf)).astype(np.float32)
    return op_out, valid.astype(np.float32)


_OP_CONST, _VMASK_CONST = _proposals_np()


def _enc_kernel(f3, f2, f1, f0, vmask, wenc, benc, lng, lnb, wcls, bcls,
                om_out, sc_out):
    i = pl.program_id(1)
    ft = jnp.where(
        i < 1, f3[0],
        jnp.where(i < 5, f2[0], jnp.where(i < 21, f1[0], f0[0])))
    ft = ft * vmask[...]
    x = lax.dot_general(ft, wenc[...], (((0,), (0,)), ((), ())),
                        preferred_element_type=jnp.float32)
    x = x + benc[...]
    mu = jnp.mean(x, axis=1, keepdims=True)
    d = x - mu
    var = jnp.mean(d * d, axis=1, keepdims=True)
    om = d / jnp.sqrt(var + 1e-5) * lng[...] + lnb[...]
    cls = jnp.dot(om, wcls[...], preferred_element_type=jnp.float32) + bcls[...]
    scol = jnp.max(cls, axis=1, keepdims=True)
    ident = (lax.broadcasted_iota(jnp.int32, (_TBLK, _TBLK), 0)
             == lax.broadcasted_iota(jnp.int32, (_TBLK, _TBLK), 1)
             ).astype(jnp.float32)
    srow = lax.dot_general(scol, ident, (((0,), (0,)), ((), ())),
                           precision=lax.Precision.HIGHEST,
                           preferred_element_type=jnp.float32)
    u = lax.bitcast_convert_type(srow, jnp.int32)
    key = u ^ (lax.shift_right_arithmetic(u, 31) & jnp.int32(0x7FFFFFFF))
    om_out[...] = om.reshape(1, _TBLK, _HID)
    sc_out[...] = key.reshape(1, 1, _TBLK)


def _run_enc(f3, f2, f1, f0, vmask, wenc, benc, lng, lnb, wclsp, bclsp):
    grid = (2, _GRID_I)
    fspec = lambda off, n: pl.BlockSpec(
        (1, _HID, _TBLK), lambda b, i: (b, 0, jnp.clip(i - off, 0, n - 1)))
    out_shapes = (
        jax.ShapeDtypeStruct((2, _L_TOT, _HID), jnp.float32),
        jax.ShapeDtypeStruct((2, 1, _L_TOT), jnp.int32),
    )
    return pl.pallas_call(
        _enc_kernel,
        grid=grid,
        in_specs=[
            fspec(0, 1), fspec(1, 4), fspec(5, 16), fspec(21, 64),
            pl.BlockSpec((1, _TBLK), lambda b, i: (0, i)),
            pl.BlockSpec((_HID, _HID), lambda b, i: (0, 0)),
            pl.BlockSpec((1, _HID), lambda b, i: (0, 0)),
            pl.BlockSpec((1, _HID), lambda b, i: (0, 0)),
            pl.BlockSpec((1, _HID), lambda b, i: (0, 0)),
            pl.BlockSpec((_HID, 128), lambda b, i: (0, 0)),
            pl.BlockSpec((1, 128), lambda b, i: (0, 0)),
        ],
        out_specs=[
            pl.BlockSpec((1, _TBLK, _HID), lambda b, i: (b, i, 0)),
            pl.BlockSpec((1, 1, _TBLK), lambda b, i: (b, 0, i)),
        ],
        out_shape=out_shapes,
        compiler_params=pltpu.CompilerParams(
            dimension_semantics=("arbitrary", "arbitrary")),
    )(f3, f2, f1, f0, vmask, wenc, benc, lng, lnb, wclsp, bclsp)


_LB = _L_TOT
_CHUNK = 1360
_NV = 85
_SELCAP = 1024
_RPAD = 1024
_I32MIN = -(2 ** 31)


def _tk_body(sc_hbm, om_hbm, tgt_hbm, idx_hbm,
             keys, hist, cvec, cnts, selk, selg, allk,
             outp, obuf, idxl, idxg, gbuf,
             sh_hist, sh_cnt, sh_keys, sh_outall, dsem):
    c = lax.axis_index("c")
    s = lax.axis_index("s")
    lane = lax.iota(jnp.int32, 16)
    zeros16 = jnp.zeros((16,), jnp.int32)

    base = c * _LB + s * _CHUNK
    pltpu.sync_copy(sc_hbm.at[pl.ds(base, _CHUNK)], keys)

    prefix = jnp.int32(0)
    need = jnp.int32(_NQ)
    for p in range(8):
        shift = 28 - 4 * p
        pm = jnp.int32(0) if p == 0 else jnp.int32(-(1 << (shift + 4)))
        for r in range(16):
            hist[pl.ds(16 * r, 16)] = zeros16

        def hbody(j, carry, shift=shift, pm=pm, p=p):
            k = keys[pl.ds(j * 16, 16)]
            act = (k & pm) == (carry & pm)
            dig = lax.shift_right_arithmetic(k, shift) & 15
            if p == 0:
                dig = dig ^ 8
            slot = dig * 16 + lane
            cur = plsc.load_gather(hist, [slot])
            plsc.store_scatter(hist, [slot], cur + jnp.where(act, 1, 0))
            return carry

        lax.fori_loop(0, _NV, hbody, prefix)
        tot = zeros16
        for j in range(16):
            tot = tot + plsc.load_gather(hist, [lane * 16 + j])
        cvec[...] = tot
        pltpu.sync_copy(cvec, sh_hist.at[pl.ds(p * 256 + s * 16, 16)])
        plsc.subcore_barrier()
        pltpu.sync_copy(sh_hist.at[pl.ds(p * 256, 256)], cnts)
        tot = zeros16
        for t in range(16):
            tot = tot + cnts[pl.ds(16 * t, 16)]
        cnt_ge = lax.rev(plsc.cumsum(lax.rev(tot, (0,))), (0,))
        maskv = cnt_ge >= need
        npop = plsc.all_reduce_population_count(maskv)
        cgt = jnp.sum(jnp.where(lane == npop, cnt_ge, 0))
        need = need - cgt
        dstar = jnp.max(npop - 1)
        if p == 0:
            dstar = dstar ^ 8
        prefix = prefix | (dstar << shift)

    thr = prefix
    need_eq = need

    def cntbody(j, carry):
        ngt, neq = carry
        k = keys[pl.ds(j * 16, 16)]
        ngt = ngt + plsc.all_reduce_population_count(k > thr)
        neq = neq + plsc.all_reduce_population_count(k == thr)
        return (ngt, neq)

    ngt_l, neq_l = lax.fori_loop(0, _NV, cntbody, (zeros16, zeros16))
    cvec[...] = (jnp.where(lane == 0, ngt_l, 0)
                 + jnp.where(lane == 1, neq_l, 0))
    pltpu.sync_copy(cvec, sh_cnt.at[pl.ds(s * 16, 16)])
    plsc.subcore_barrier()
    pltpu.sync_copy(sh_cnt, cnts)
    gt_col = plsc.load_gather(cnts, [lane * 16])
    eq_col = plsc.load_gather(cnts, [lane * 16 + 1])
    cs_gt = plsc.cumsum(gt_col)
    cs_eq = plsc.cumsum(eq_col)
    pre_eq = jnp.sum(jnp.where(lane == s - 1, cs_eq, 0))
    tot_gt = jnp.sum(jnp.where(lane == 15, cs_gt, 0))

    for r in range(_SELCAP // 16):
        selk[pl.ds(16 * r, 16)] = zeros16 + jnp.int32(_I32MIN)
    for r in range(64):
        outp[pl.ds(16 * r, 16)] = zeros16

    def cbody(j, carry):
        cgt_loc, ceq_loc = carry
        k = keys[pl.ds(j * 16, 16)]
        gidx = s * _CHUNK + j * 16 + lane
        g = k > thr
        e = k == thr
        gi = jnp.where(g, 1, 0)
        pos = jnp.where(g, cgt_loc + plsc.cumsum(gi) - 1, 0)
        plsc.store_scatter(selk, [pos], k, mask=g)
        plsc.store_scatter(selg, [pos], gidx, mask=g)
        ei = jnp.where(e, 1, 0)
        ger = pre_eq + ceq_loc + plsc.cumsum(ei) - 1
        keep = e & (ger < need_eq)
        orank = jnp.where(keep, tot_gt + ger, 0)
        plsc.store_scatter(outp, [orank], gidx, mask=keep)
        return (cgt_loc + jnp.sum(gi), ceq_loc + jnp.sum(ei))

    lax.fori_loop(0, _NV, cbody, (jnp.int32(0), jnp.int32(0)))

    pltpu.sync_copy(selk, sh_keys.at[pl.ds(s * _SELCAP, _SELCAP)])
    plsc.subcore_barrier()
    pltpu.sync_copy(sh_keys, allk)

    nv_col = lax.shift_right_arithmetic(gt_col + 15, 4)
    n_gt_loc = jnp.sum(jnp.where(lane == s, gt_col, 0))

    def rbody(e, carry):
        lo = e & 15
        be = e - lo
        kv = selk[pl.ds(be, 16)]
        ke = jnp.sum(jnp.where(lane == lo, kv, 0))
        gv = selg[pl.ds(be, 16)]
        ge = jnp.sum(jnp.where(lane == lo, gv, 0))
        total = zeros16
        for w in range(16):
            nvw = jnp.sum(jnp.where(lane == w, nv_col, 0))

            def ib(jv, t, w=w):
                ak = allk[pl.ds(w * _SELCAP + jv * 16, 16)]
                t = t + plsc.all_reduce_population_count(ak > ke)
                eqc = plsc.all_reduce_population_count(ak == ke)
                return t + jnp.where(w < s, eqc, 0)

            total = lax.fori_loop(0, nvw, ib, total)
        nve = lax.shift_right_arithmetic(e + 15, 4)

        def ob(jv, t):
            ok = selk[pl.ds(jv * 16, 16)]
            valid = (jv * 16 + lane) < e
            return t + plsc.all_reduce_population_count((ok == ke) & valid)

        total = lax.fori_loop(0, nve, ob, total)
        rank = jnp.max(total)
        plsc.store_scatter(outp, [lane * 0 + rank], lane * 0 + ge,
                           mask=lane == 0)
        return carry

    lax.fori_loop(0, n_gt_loc, rbody, 0)

    pltpu.sync_copy(outp, sh_outall.at[pl.ds(s * _RPAD, _RPAD)])
    plsc.subcore_barrier()
    a0, a1, a2, a3 = zeros16, zeros16, zeros16, zeros16
    for t in range(16):
        pltpu.sync_copy(sh_outall.at[pl.ds(t * _RPAD + 64 * s, 64)], obuf)
        a0 = a0 + obuf[pl.ds(0, 16)]
        a1 = a1 + obuf[pl.ds(16, 16)]
        a2 = a2 + obuf[pl.ds(32, 16)]
        a3 = a3 + obuf[pl.ds(48, 16)]
    for r, a in enumerate((a0, a1, a2, a3)):
        idxl[pl.ds(16 * r, 16)] = a
        idxg[pl.ds(16 * r, 16)] = a + c * _LB
    pltpu.async_copy(om_hbm.at[idxg], gbuf, dsem).wait()
    pltpu.sync_copy(gbuf, tgt_hbm.at[c, pl.ds(64 * s, 64)])
    pltpu.sync_copy(idxl, idx_hbm.at[c, pl.ds(64 * s, 64)])


def _run_topk_gather(scores_flat, om_flat):
    mesh = plsc.VectorSubcoreMesh(core_axis_name="c", subcore_axis_name="s")
    scratch = [
        pltpu.VMEM((_CHUNK,), jnp.int32),
        pltpu.VMEM((256,), jnp.int32),
        pltpu.VMEM((16,), jnp.int32),
        pltpu.VMEM((256,), jnp.int32),
        pltpu.VMEM((_SELCAP,), jnp.int32),
        pltpu.VMEM((_SELCAP,), jnp.int32),
        pltpu.VMEM((16 * _SELCAP,), jnp.int32),
        pltpu.VMEM((_RPAD,), jnp.int32),
        pltpu.VMEM((64,), jnp.int32),
        pltpu.VMEM((64,), jnp.int32),
        pltpu.VMEM((64,), jnp.int32),
        pltpu.VMEM((64, _HID), jnp.float32),
        pltpu.VMEM_SHARED((8 * 256,), jnp.int32),
        pltpu.VMEM_SHARED((256,), jnp.int32),
        pltpu.VMEM_SHARED((16 * _SELCAP,), jnp.int32),
        pltpu.VMEM_SHARED((16 * _RPAD,), jnp.int32),
        pltpu.SemaphoreType.DMA,
    ]
    kfn = pl.kernel(
        _tk_body,
        out_type=(
            jax.ShapeDtypeStruct((2, _RPAD, _HID), jnp.float32),
            jax.ShapeDtypeStruct((2, _RPAD), jnp.int32),
        ),
        mesh=mesh,
        scratch_types=scratch,
        compiler_params=pltpu.CompilerParams(needs_layout_passes=False),
    )
    return kfn(scores_flat, om_flat)


_LVL_OFF = (0, 256, 1280, 5376, 21760)
_LVL_W = (16, 32, 64, 128)


def _mlp_kernel(tgt, idxp, w1, b1, w2, b2, w3p, b3p, ref_out):
    t = tgt[0]
    h = jnp.maximum(
        jnp.dot(t, w1[...], preferred_element_type=jnp.float32) + b1[...], 0.0)
    h = jnp.maximum(
        jnp.dot(h, w2[...], preferred_element_type=jnp.float32) + b2[...], 0.0)
    delta = jnp.dot(h, w3p[...], preferred_element_type=jnp.float32) + b3p[...]

    idx = idxp[0]
    px = jnp.zeros(idx.shape, jnp.float32)
    py = jnp.zeros(idx.shape, jnp.float32)
    pw = jnp.zeros(idx.shape, jnp.float32)
    for l in range(4):
        w = _LVL_W[l]
        local = idx - _LVL_OFF[l]
        hh = lax.shift_right_arithmetic(local, l + 4)
        ww = local - hh * w
        m = (idx >= _LVL_OFF[l]) & (idx < _LVL_OFF[l + 1])
        px = px + jnp.where(m, (ww.astype(jnp.float32) + 0.5) / w, 0.0)
        py = py + jnp.where(m, (hh.astype(jnp.float32) + 0.5) / w, 0.0)
        pw = pw + jnp.where(m, jnp.float32(0.05 * (2.0 ** l)), 0.0)
    valid = (px > 0.01) & (px < 0.99) & (py > 0.01) & (py < 0.99)
    big = jnp.float32(3e38)
    lx = jnp.where(valid, jnp.log(px / (1.0 - px)), big)
    ly = jnp.where(valid, jnp.log(py / (1.0 - py)), big)
    lw = jnp.where(valid, jnp.log(pw / (1.0 - pw)), big)
    opmat = jnp.concatenate([lx, ly, lw, lw], axis=0)
    eye = (lax.broadcasted_iota(jnp.int32, (4, 8), 0)
           == lax.broadcasted_iota(jnp.int32, (4, 8), 1)).astype(jnp.float32)
    op8 = lax.dot_general(opmat, eye, (((0,), (0,)), ((), ())),
                          precision=lax.Precision.HIGHEST,
                          preferred_element_type=jnp.float32)
    op8 = jnp.where(op8 > jnp.float32(1e29), jnp.float32(np.inf), op8)
    ref_out[...] = (delta + op8).reshape(1, _RPAD, 8)


def _run_mlp(tgt_pad, idx_pad, w1, b1, w2, b2, w3p, b3p):
    return pl.pallas_call(
        _mlp_kernel,
        grid=(2,),
        in_specs=[
            pl.BlockSpec((1, _RPAD, _HID), lambda b: (b, 0, 0)),
            pl.BlockSpec((1, 1, _RPAD), lambda b: (b, 0, 0)),
            pl.BlockSpec((_HID, _HID), lambda b: (0, 0)),
            pl.BlockSpec((1, _HID), lambda b: (0, 0)),
            pl.BlockSpec((_HID, _HID), lambda b: (0, 0)),
            pl.BlockSpec((1, _HID), lambda b: (0, 0)),
            pl.BlockSpec((_HID, 8), lambda b: (0, 0)),
            pl.BlockSpec((1, 8), lambda b: (0, 0)),
        ],
        out_specs=pl.BlockSpec((1, _RPAD, 8), lambda b: (b, 0, 0)),
        out_shape=jax.ShapeDtypeStruct((2, _RPAD, 8), jnp.float32),
    )(tgt_pad, idx_pad, w1, b1, w2, b2, w3p, b3p)


def kernel(feat0, feat1, feat2, feat3, W_enc, b_enc, ln_g, ln_b, W_cls, b_cls,
           Wb1, bb1, Wb2, bb2, Wb3, bb3):
    f3 = feat3.reshape(2, _HID, 256)
    f2 = feat2.reshape(2, _HID, 1024)
    f1 = feat1.reshape(2, _HID, 4096)
    f0 = feat0.reshape(2, _HID, 16384)
    vmask = jnp.asarray(_VMASK_CONST).reshape(1, _L_TOT)
    wclsp = jnp.pad(W_cls, ((0, 0), (0, 128 - _NCLS)))
    bclsp = jnp.pad(b_cls, (0, 128 - _NCLS), constant_values=_NEG).reshape(1, 128)

    om, sc = _run_enc(f3, f2, f1, f0, vmask, W_enc, b_enc.reshape(1, _HID),
                      ln_g.reshape(1, _HID), ln_b.reshape(1, _HID), wclsp, bclsp)
    scores_flat = sc.reshape(2 * _L_TOT)
    om_flat = om.reshape(2 * _L_TOT, _HID)

    tgt_pad, idx_pad = _run_topk_gather(scores_flat, om_flat)

    refp = _run_mlp(tgt_pad, idx_pad.reshape(2, 1, _RPAD), Wb1,
                    bb1.reshape(1, _HID), Wb2, bb2.reshape(1, _HID),
                    jnp.pad(Wb3, ((0, 0), (0, 4))),
                    jnp.pad(bb3, (0, 4)).reshape(1, 8))
    tgt = tgt_pad[:, :_NQ]
    refpoint = refp[:, :_NQ, :4]
    return lax.stop_gradient(tgt), lax.stop_gradient(refpoint)
